# Initial kernel scaffold; baseline (speedup 1.0000x reference)
import jax
import jax.numpy as jnp
from jax.experimental import pallas as pl


def kernel(x, edge_index, batch, v):
    raise NotImplementedError("write your pallas kernel here")


# speedup vs baseline: 78.3323x; 78.3323x over previous
import functools

import jax
import jax.numpy as jnp
from jax import lax
from jax.experimental import pallas as pl
from jax.experimental.pallas import tpu as pltpu
from jax.experimental.pallas import tpu_sc as plsc

_T = 16
_S = 16
_ST = _S * _T
_G = 64
_N = 10000
_E = 160000
_F = 128

_NH_BLK = 2000
_CHUNK = 128
_NCHUNK = _E // _CHUNK
_NTILES = 32
_CPT = -(-_NCHUNK // _NTILES)

_R_BLK = 2048
_R_TOT = 172032


def _nh_body(x_ref, v_ref, o_ref):
    o_ref[...] = lax.dot_general(
        x_ref[...], v_ref[...], (((1,), (0,)), ((), ())),
        preferred_element_type=jnp.float32)


def _compute_nh(x, v):
    return pl.pallas_call(
        _nh_body,
        grid=(_N // _NH_BLK,),
        in_specs=[
            pl.BlockSpec((_NH_BLK, _F), lambda i: (i, 0)),
            pl.BlockSpec((_F, _T), lambda i: (0, 0)),
        ],
        out_specs=pl.BlockSpec((_NH_BLK, _T), lambda i: (i, 0)),
        out_shape=jax.ShapeDtypeStruct((_N, _T), jnp.float32),
    )(x, v)


def _sc_edge_gather(nh, i0, i1, batch):
    mesh = plsc.VectorSubcoreMesh(core_axis_name="c", subcore_axis_name="s")

    @functools.partial(
        pl.kernel,
        mesh=mesh,
        compiler_params=pltpu.CompilerParams(
            needs_layout_passes=False, use_tc_tiling_on_sc=False),
        out_type=[
            jax.ShapeDtypeStruct((_T, _E), jnp.float32),
            jax.ShapeDtypeStruct((_E,), jnp.float32),
        ],
        scratch_types=[
            pltpu.VMEM((_CHUNK,), jnp.int32),
            pltpu.VMEM((_CHUNK,), jnp.int32),
            pltpu.VMEM((_CHUNK, _T), jnp.float32),
            pltpu.VMEM((_CHUNK, _T), jnp.float32),
            pltpu.VMEM((_T, _CHUNK), jnp.float32),
            pltpu.VMEM((_CHUNK,), jnp.float32),
            pltpu.VMEM((_N,), jnp.int32),
            pltpu.SemaphoreType.DMA,
            pltpu.SemaphoreType.DMA,
        ],
    )
    def k(nh_hbm, i0_hbm, i1_hbm, b_hbm, eht_hbm, gw_hbm,
          idx0_v, idx1_v, ra, rb, et, gwb, batch_v, sem_a, sem_b):
        wid = lax.axis_index("s") * 2 + lax.axis_index("c")
        pltpu.sync_copy(b_hbm, batch_v)
        lanes = lax.iota(jnp.int32, 16)

        def chunk(kk, carry):
            cid = wid + kk * _NTILES

            @pl.when(cid < _NCHUNK)
            def _():
                cb = cid * _CHUNK
                pltpu.sync_copy(i0_hbm.at[pl.ds(cb, _CHUNK)], idx0_v)
                pltpu.sync_copy(i1_hbm.at[pl.ds(cb, _CHUNK)], idx1_v)
                cp_a = pltpu.async_copy(nh_hbm.at[idx0_v], ra, sem_a)
                cp_b = pltpu.async_copy(nh_hbm.at[idx1_v], rb, sem_b)
                cp_a.wait()
                cp_b.wait()

                def gbody(i, c):
                    idx16 = idx0_v[pl.ds(i * 16, 16)]
                    g16 = plsc.load_gather(batch_v, [idx16])
                    gwb[pl.ds(i * 16, 16)] = -(g16.astype(jnp.float32) + 0.25)
                    return c
                lax.fori_loop(0, _CHUNK // 16, gbody, 0)

                def ebody(r, c):
                    eh = jnp.maximum(ra[r, :], rb[r, :])
                    plsc.store_scatter(et, [lanes, jnp.full((16,), r, jnp.int32)], eh)
                    return c
                lax.fori_loop(0, _CHUNK, ebody, 0)

                pltpu.sync_copy(et, eht_hbm.at[:, pl.ds(cb, _CHUNK)])
                pltpu.sync_copy(gwb, gw_hbm.at[pl.ds(cb, _CHUNK)])
            return carry
        lax.fori_loop(0, _CPT, chunk, 0)

    return k(nh, i0, i1, batch)


def _ecc_body(y_ref, gw_ref, o_ref):
    yb = y_ref[...]
    yt = jnp.concatenate([yb] * _S, axis=0)
    srow = lax.broadcasted_iota(jnp.int32, (_ST, _R_BLK), 0) >> 4
    lin = srow.astype(jnp.float32) * jnp.float32(2.0 / 15.0) - 1.0
    sig = 1.0 / (1.0 + jnp.exp(200.0 * (yt - lin)))

    gw = gw_ref[0:1, :]
    gbc = jnp.broadcast_to(gw, (_G, _R_BLK))
    iot = lax.broadcasted_iota(jnp.int32, (_G, _R_BLK), 0)
    oh = jnp.where(jnp.abs(gbc).astype(jnp.int32) == iot,
                   jnp.sign(gbc), 0.0)
    d = lax.dot_general(
        sig.astype(jnp.bfloat16), oh.astype(jnp.bfloat16),
        (((1,), (1,)), ((), ())), preferred_element_type=jnp.float32)

    @pl.when(pl.program_id(0) == 0)
    def _():
        o_ref[...] = d

    @pl.when(pl.program_id(0) != 0)
    def _():
        o_ref[...] += d


def _compute_ecc(yt, gw8):
    return pl.pallas_call(
        _ecc_body,
        grid=(_R_TOT // _R_BLK,),
        in_specs=[
            pl.BlockSpec((_T, _R_BLK), lambda i: (0, i)),
            pl.BlockSpec((8, _R_BLK), lambda i: (0, i)),
        ],
        out_specs=pl.BlockSpec((_ST, _G), lambda i: (0, 0)),
        out_shape=jax.ShapeDtypeStruct((_ST, _G), jnp.float32),
    )(yt, gw8)


def kernel(x, edge_index, batch, v):
    nh = _compute_nh(x, v)
    eht, gwe = _sc_edge_gather(nh, edge_index[0], edge_index[1], batch)

    yt = jnp.concatenate([nh.T, eht], axis=1)
    gw = jnp.concatenate([batch.astype(jnp.float32) + 0.25, gwe])
    pad = _R_TOT - (_N + _E)
    ytp = jnp.pad(yt, ((0, 0), (0, pad)))
    gw8 = jnp.broadcast_to(jnp.pad(gw, (0, pad))[None, :], (8, _R_TOT))

    out2 = _compute_ecc(ytp, gw8)
    return out2.T.reshape(_G, _S, _T)


# speedup vs baseline: 101.7725x; 1.2992x over previous
import functools

import jax
import jax.numpy as jnp
from jax import lax
from jax.experimental import pallas as pl
from jax.experimental.pallas import tpu as pltpu
from jax.experimental.pallas import tpu_sc as plsc

_T = 16
_S = 16
_ST = _S * _T
_G = 64
_N = 10000
_E = 160000
_F = 128

_NH_BLK = 2000
_CHUNK = 128
_NCHUNK = _E // _CHUNK
_NTILES = 32
_CPT = -(-_NCHUNK // _NTILES)

_R_BLK = 2048
_R_TOT = 172032


def _nh_body(x_ref, v_ref, o_ref):
    o_ref[...] = lax.dot_general(
        x_ref[...], v_ref[...], (((1,), (0,)), ((), ())),
        preferred_element_type=jnp.float32)


def _compute_nh(x, v):
    return pl.pallas_call(
        _nh_body,
        grid=(_N // _NH_BLK,),
        in_specs=[
            pl.BlockSpec((_NH_BLK, _F), lambda i: (i, 0)),
            pl.BlockSpec((_F, _T), lambda i: (0, 0)),
        ],
        out_specs=pl.BlockSpec((_NH_BLK, _T), lambda i: (i, 0)),
        out_shape=jax.ShapeDtypeStruct((_N, _T), jnp.float32),
    )(x, v)


def _sc_edge_gather(nh, ei, batch):
    mesh = plsc.VectorSubcoreMesh(core_axis_name="c", subcore_axis_name="s")

    @functools.partial(
        pl.kernel,
        mesh=mesh,
        compiler_params=pltpu.CompilerParams(
            needs_layout_passes=False, use_tc_tiling_on_sc=False),
        out_type=jax.ShapeDtypeStruct((_T + 1, _E), jnp.float32),
        scratch_types=[
            pltpu.VMEM((2, _CHUNK), jnp.int32),
            pltpu.VMEM((2, _CHUNK), jnp.int32),
            pltpu.VMEM((_CHUNK, _T), jnp.float32),
            pltpu.VMEM((_CHUNK, _T), jnp.float32),
            pltpu.VMEM((_CHUNK, _T), jnp.float32),
            pltpu.VMEM((_CHUNK, _T), jnp.float32),
            pltpu.VMEM((_T + 1, _CHUNK), jnp.float32),
            pltpu.VMEM((_T + 1, _CHUNK), jnp.float32),
            pltpu.VMEM((_N,), jnp.int32),
            pltpu.SemaphoreType.DMA,
            pltpu.SemaphoreType.DMA,
            pltpu.SemaphoreType.DMA,
            pltpu.SemaphoreType.DMA,
            pltpu.SemaphoreType.DMA,
            pltpu.SemaphoreType.DMA,
            pltpu.SemaphoreType.DMA,
        ],
    )
    def k(nh_hbm, ei_hbm, b_hbm, out_hbm,
          idx0, idx1, ra0, rb0, ra1, rb1, et0, et1, batch_v,
          sga0, sgb0, so0, sga1, sgb1, so1, si):
        wid = lax.axis_index("s") * 2 + lax.axis_index("c")
        pltpu.sync_copy(b_hbm, batch_v)
        lanes = lax.iota(jnp.int32, 16)
        slots = ((idx0, ra0, rb0, et0, sga0, sgb0, so0),
                 (idx1, ra1, rb1, et1, sga1, sgb1, so1))

        for b in (0, 1):
            idx, ra, rb, _, sga, sgb, _ = slots[b]
            cb = (wid + b * _NTILES) * _CHUNK
            pltpu.async_copy(ei_hbm.at[:, pl.ds(cb, _CHUNK)], idx, si).wait()
            pltpu.async_copy(nh_hbm.at[idx.at[0]], ra, sga)
            pltpu.async_copy(nh_hbm.at[idx.at[1]], rb, sgb)

        def step(i, carry):
            for b in (0, 1):
                idx, ra, rb, et, sga, sgb, so = slots[b]
                kk = 2 * i + b
                cid = wid + kk * _NTILES

                @pl.when(cid < _NCHUNK)
                def _():
                    cb = cid * _CHUNK
                    pltpu.make_async_copy(nh_hbm.at[idx.at[0]], ra, sga).wait()
                    pltpu.make_async_copy(nh_hbm.at[idx.at[1]], rb, sgb).wait()

                    @pl.when(kk >= 2)
                    def _():
                        pltpu.make_async_copy(
                            et, out_hbm.at[:, pl.ds(cb, _CHUNK)], so).wait()

                    def compute(i16, c):
                        g16 = plsc.load_gather(batch_v, [idx[0, pl.ds(i16 * 16, 16)]])
                        et[16, pl.ds(i16 * 16, 16)] = -(g16.astype(jnp.float32) + 0.25)
                        for j in range(16):
                            r = i16 * 16 + j
                            eh = jnp.maximum(ra[r, :], rb[r, :])
                            plsc.store_scatter(
                                et, [lanes, jnp.full((16,), r, jnp.int32)], eh)
                        return c
                    lax.fori_loop(0, _CHUNK // 16, compute, 0)

                    pltpu.async_copy(et, out_hbm.at[:, pl.ds(cb, _CHUNK)], so)

                    nid = cid + 2 * _NTILES

                    @pl.when(nid < _NCHUNK)
                    def _():
                        ncb = nid * _CHUNK
                        pltpu.async_copy(
                            ei_hbm.at[:, pl.ds(ncb, _CHUNK)], idx, si).wait()
                        pltpu.async_copy(nh_hbm.at[idx.at[0]], ra, sga)
                        pltpu.async_copy(nh_hbm.at[idx.at[1]], rb, sgb)
            return carry
        lax.fori_loop(0, _CPT // 2, step, 0)

        for b in (0, 1):
            _, _, _, et, _, _, so = slots[b]
            cb = (wid + b * _NTILES) * _CHUNK
            pltpu.make_async_copy(et, out_hbm.at[:, pl.ds(cb, _CHUNK)], so).wait()

    return k(nh, ei, batch)


def _ecc_body(y_ref, o_ref):
    yb = y_ref[0:16, :]
    gw = y_ref[16:17, :]
    yt = jnp.concatenate([yb] * _S, axis=0)
    srow = lax.broadcasted_iota(jnp.int32, (_ST, _R_BLK), 0) >> 4
    lin = srow.astype(jnp.float32) * jnp.float32(2.0 / 15.0) - 1.0
    sig = 0.5 + 0.5 * jnp.tanh(100.0 * (lin - yt))

    gbc = jnp.broadcast_to(gw, (_G, _R_BLK))
    iot = lax.broadcasted_iota(jnp.int32, (_G, _R_BLK), 0)
    oh = jnp.where(jnp.abs(gbc).astype(jnp.int32) == iot,
                   jnp.sign(gbc), 0.0)
    d = lax.dot_general(
        sig.astype(jnp.bfloat16), oh.astype(jnp.bfloat16),
        (((1,), (1,)), ((), ())), preferred_element_type=jnp.float32)

    @pl.when(pl.program_id(0) == 0)
    def _():
        o_ref[...] = d

    @pl.when(pl.program_id(0) != 0)
    def _():
        o_ref[...] += d


def _compute_ecc(y17):
    return pl.pallas_call(
        _ecc_body,
        grid=(_R_TOT // _R_BLK,),
        in_specs=[pl.BlockSpec((_T + 1, _R_BLK), lambda i: (0, i))],
        out_specs=pl.BlockSpec((_ST, _G), lambda i: (0, 0)),
        out_shape=jax.ShapeDtypeStruct((_ST, _G), jnp.float32),
    )(y17)


def kernel(x, edge_index, batch, v):
    nh = _compute_nh(x, v)
    e17 = _sc_edge_gather(nh, edge_index, batch)

    n17 = jnp.concatenate(
        [nh.T, (batch.astype(jnp.float32) + 0.25)[None, :]], axis=0)
    y17 = jnp.concatenate([n17, e17], axis=1)
    y17p = jnp.pad(y17, ((0, 0), (0, _R_TOT - (_N + _E))))

    out2 = _compute_ecc(y17p)
    return out2.T.reshape(_G, _S, _T)


# speedup vs baseline: 132.6729x; 1.3036x over previous
import functools

import jax
import jax.numpy as jnp
from jax import lax
from jax.experimental import pallas as pl
from jax.experimental.pallas import tpu as pltpu
from jax.experimental.pallas import tpu_sc as plsc

_T = 16
_S = 16
_ST = _S * _T
_G = 64
_N = 10000
_E = 160000
_F = 128

_NH_BLK = 2000
_CHUNK = 128
_NCHUNK = _E // _CHUNK
_NTILES = 32
_CPT = 40

_NP = 10240
_NODE_BLK = 2048
_EDGE_BLK = 3200


def _nh_body(x_ref, v_ref, o_ref):
    o_ref[...] = lax.dot_general(
        x_ref[...], v_ref[...], (((1,), (0,)), ((), ())),
        preferred_element_type=jnp.float32)


def _compute_nh(x, v):
    return pl.pallas_call(
        _nh_body,
        grid=(_N // _NH_BLK,),
        in_specs=[
            pl.BlockSpec((_NH_BLK, _F), lambda i: (i, 0)),
            pl.BlockSpec((_F, _T), lambda i: (0, 0)),
        ],
        out_specs=pl.BlockSpec((_NH_BLK, _T), lambda i: (i, 0)),
        out_shape=jax.ShapeDtypeStruct((_N, _T), jnp.float32),
    )(x, v)


def _sc_edge_gather(nh, ei, batch):
    mesh = plsc.VectorSubcoreMesh(core_axis_name="c", subcore_axis_name="s")

    @functools.partial(
        pl.kernel,
        mesh=mesh,
        compiler_params=pltpu.CompilerParams(
            needs_layout_passes=False, use_tc_tiling_on_sc=False),
        out_type=jax.ShapeDtypeStruct((_T + 1, _E), jnp.float32),
        scratch_types=[
            pltpu.VMEM((2, _CPT * _CHUNK), jnp.int32),
            pltpu.VMEM((_CHUNK, _T), jnp.float32),
            pltpu.VMEM((_CHUNK, _T), jnp.float32),
            pltpu.VMEM((_CHUNK, _T), jnp.float32),
            pltpu.VMEM((_CHUNK, _T), jnp.float32),
            pltpu.VMEM((_T + 1, _CHUNK), jnp.float32),
            pltpu.VMEM((_T + 1, _CHUNK), jnp.float32),
            pltpu.VMEM((_N,), jnp.int32),
            pltpu.SemaphoreType.DMA,
            pltpu.SemaphoreType.DMA,
            pltpu.SemaphoreType.DMA,
            pltpu.SemaphoreType.DMA,
            pltpu.SemaphoreType.DMA,
            pltpu.SemaphoreType.DMA,
            pltpu.SemaphoreType.DMA,
        ],
    )
    def k(nh_hbm, ei_hbm, b_hbm, out_hbm,
          idx_all, ra0, rb0, ra1, rb1, et0, et1, batch_v,
          sga0, sgb0, so0, sga1, sgb1, so1, si):
        wid = lax.axis_index("s") * 2 + lax.axis_index("c")
        start = 39 * wid + jnp.maximum(wid - 30, 0)
        cnt = jnp.where(wid >= 30, 40, 39)
        base = start * _CHUNK
        pltpu.async_copy(ei_hbm.at[:, pl.ds(base, _CPT * _CHUNK)], idx_all, si)
        pltpu.sync_copy(b_hbm, batch_v)
        pltpu.make_async_copy(
            ei_hbm.at[:, pl.ds(base, _CPT * _CHUNK)], idx_all, si).wait()
        lanes = lax.iota(jnp.int32, 16)
        slots = ((ra0, rb0, et0, sga0, sgb0, so0),
                 (ra1, rb1, et1, sga1, sgb1, so1))

        def gath(kk, ra, rb, sga, sgb):
            off = kk * _CHUNK
            pltpu.async_copy(nh_hbm.at[idx_all.at[0, pl.ds(off, _CHUNK)]], ra, sga)
            pltpu.async_copy(nh_hbm.at[idx_all.at[1, pl.ds(off, _CHUNK)]], rb, sgb)

        for b in (0, 1):
            ra, rb, _, sga, sgb, _ = slots[b]
            gath(jnp.int32(b), ra, rb, sga, sgb)

        def step(i, carry):
            for b in (0, 1):
                ra, rb, et, sga, sgb, so = slots[b]
                kk = 2 * i + b

                @pl.when(kk < cnt)
                def _():
                    cb = (start + kk) * _CHUNK
                    off = kk * _CHUNK
                    pltpu.make_async_copy(
                        nh_hbm.at[idx_all.at[0, pl.ds(off, _CHUNK)]], ra, sga).wait()
                    pltpu.make_async_copy(
                        nh_hbm.at[idx_all.at[1, pl.ds(off, _CHUNK)]], rb, sgb).wait()

                    @pl.when(kk >= 2)
                    def _():
                        pltpu.make_async_copy(
                            et, out_hbm.at[:, pl.ds(cb, _CHUNK)], so).wait()

                    def compute(i16, c):
                        g16 = plsc.load_gather(
                            batch_v, [idx_all[0, pl.ds(off + i16 * 16, 16)]])
                        et[16, pl.ds(i16 * 16, 16)] = -(g16.astype(jnp.float32) + 0.25)
                        for j in range(16):
                            r = i16 * 16 + j
                            eh = jnp.maximum(ra[r, :], rb[r, :])
                            plsc.store_scatter(
                                et, [lanes, jnp.full((16,), r, jnp.int32)], eh)
                        return c
                    lax.fori_loop(0, _CHUNK // 16, compute, 0)

                    pltpu.async_copy(et, out_hbm.at[:, pl.ds(cb, _CHUNK)], so)

                    @pl.when(kk + 2 < cnt)
                    def _():
                        gath(kk + 2, ra, rb, sga, sgb)
            return carry
        lax.fori_loop(0, _CPT // 2, step, 0)

        for b in (0, 1):
            _, _, et, _, _, so = slots[b]
            pltpu.make_async_copy(
                et, out_hbm.at[:, pl.ds(base, _CHUNK)], so).wait()

    return k(nh, ei, batch)


def _ecc_body(blk, y_ref, o_ref):
    yb = y_ref[0:16, :]
    gw = y_ref[16:17, :]
    yt = jnp.concatenate([yb] * _S, axis=0)
    srow = lax.broadcasted_iota(jnp.int32, (_ST, blk), 0) >> 4
    lin = srow.astype(jnp.float32) * jnp.float32(2.0 / 15.0) - 1.0
    sig = 0.5 + 0.5 * jnp.tanh(100.0 * (lin - yt))

    gbc = jnp.broadcast_to(gw, (_G, blk))
    iot = lax.broadcasted_iota(jnp.int32, (_G, blk), 0)
    oh = jnp.where(jnp.abs(gbc).astype(jnp.int32) == iot,
                   jnp.sign(gbc), 0.0)
    d = lax.dot_general(
        sig.astype(jnp.bfloat16), oh.astype(jnp.bfloat16),
        (((1,), (1,)), ((), ())), preferred_element_type=jnp.float32)

    @pl.when(pl.program_id(0) == 0)
    def _():
        o_ref[...] = d

    @pl.when(pl.program_id(0) != 0)
    def _():
        o_ref[...] += d


def _compute_ecc(y17, blk):
    n = y17.shape[1]
    return pl.pallas_call(
        functools.partial(_ecc_body, blk),
        grid=(n // blk,),
        in_specs=[pl.BlockSpec((_T + 1, blk), lambda i: (0, i))],
        out_specs=pl.BlockSpec((_ST, _G), lambda i: (0, 0)),
        out_shape=jax.ShapeDtypeStruct((_ST, _G), jnp.float32),
    )(y17)


def kernel(x, edge_index, batch, v):
    nh = _compute_nh(x, v)
    e17 = _sc_edge_gather(nh, edge_index, batch)

    n17 = jnp.concatenate(
        [nh.T, (batch.astype(jnp.float32) + 0.25)[None, :]], axis=0)
    n17p = jnp.pad(n17, ((0, 0), (0, _NP - _N)))

    out2 = _compute_ecc(n17p, _NODE_BLK) + _compute_ecc(e17, _EDGE_BLK)
    return out2.T.reshape(_G, _S, _T)


# speedup vs baseline: 154.3098x; 1.1631x over previous
import functools

import jax
import jax.numpy as jnp
from jax import lax
from jax.experimental import pallas as pl
from jax.experimental.pallas import tpu as pltpu
from jax.experimental.pallas import tpu_sc as plsc

_T = 16
_S = 16
_ST = _S * _T
_G = 64
_N = 10000
_E = 160000
_F = 128

_NH_BLK = 2000
_CHUNK = 128
_NCHUNK = _E // _CHUNK
_NTILES = 32
_CPT = 40

_NP = 10240
_NODE_BLK = 2048
_EDGE_BLK = 3200


def _nh_body(x_ref, v_ref, o_ref):
    o_ref[...] = lax.dot_general(
        x_ref[...], v_ref[...], (((1,), (0,)), ((), ())),
        preferred_element_type=jnp.float32)


def _compute_nh(x, v):
    return pl.pallas_call(
        _nh_body,
        grid=(_N // _NH_BLK,),
        in_specs=[
            pl.BlockSpec((_NH_BLK, _F), lambda i: (i, 0)),
            pl.BlockSpec((_F, _T), lambda i: (0, 0)),
        ],
        out_specs=pl.BlockSpec((_NH_BLK, _T), lambda i: (i, 0)),
        out_shape=jax.ShapeDtypeStruct((_N, _T), jnp.float32),
    )(x, v)


def _sc_edge_gather(nh, ei, batch, n_edges, cpt, base_cnt, n_extra):
    mesh = plsc.VectorSubcoreMesh(core_axis_name="c", subcore_axis_name="s")

    @functools.partial(
        pl.kernel,
        mesh=mesh,
        compiler_params=pltpu.CompilerParams(
            needs_layout_passes=False, use_tc_tiling_on_sc=False),
        out_type=jax.ShapeDtypeStruct((_T + 1, n_edges), jnp.float32),
        scratch_types=[
            pltpu.VMEM((2, cpt * _CHUNK), jnp.int32),
            pltpu.VMEM((_CHUNK, _T), jnp.float32),
            pltpu.VMEM((_CHUNK, _T), jnp.float32),
            pltpu.VMEM((_CHUNK, _T), jnp.float32),
            pltpu.VMEM((_CHUNK, _T), jnp.float32),
            pltpu.VMEM((_T + 1, _CHUNK), jnp.float32),
            pltpu.VMEM((_T + 1, _CHUNK), jnp.float32),
            pltpu.VMEM((_N,), jnp.int32),
            pltpu.SemaphoreType.DMA,
            pltpu.SemaphoreType.DMA,
            pltpu.SemaphoreType.DMA,
            pltpu.SemaphoreType.DMA,
            pltpu.SemaphoreType.DMA,
            pltpu.SemaphoreType.DMA,
            pltpu.SemaphoreType.DMA,
        ],
    )
    def k(nh_hbm, ei_hbm, b_hbm, out_hbm,
          idx_all, ra0, rb0, ra1, rb1, et0, et1, batch_v,
          sga0, sgb0, so0, sga1, sgb1, so1, si):
        wid = lax.axis_index("s") * 2 + lax.axis_index("c")
        start = base_cnt * wid + jnp.maximum(wid - (_NTILES - n_extra), 0)
        cnt = jnp.where(wid >= _NTILES - n_extra, base_cnt + 1, base_cnt)
        base = start * _CHUNK
        pltpu.async_copy(ei_hbm.at[:, pl.ds(base, cpt * _CHUNK)], idx_all, si)
        pltpu.sync_copy(b_hbm, batch_v)
        pltpu.make_async_copy(
            ei_hbm.at[:, pl.ds(base, cpt * _CHUNK)], idx_all, si).wait()
        lanes = lax.iota(jnp.int32, 16)
        slots = ((ra0, rb0, et0, sga0, sgb0, so0),
                 (ra1, rb1, et1, sga1, sgb1, so1))

        def gath(kk, ra, rb, sga, sgb):
            off = kk * _CHUNK
            pltpu.async_copy(nh_hbm.at[idx_all.at[0, pl.ds(off, _CHUNK)]], ra, sga)
            pltpu.async_copy(nh_hbm.at[idx_all.at[1, pl.ds(off, _CHUNK)]], rb, sgb)

        for b in (0, 1):
            ra, rb, _, sga, sgb, _ = slots[b]
            gath(jnp.int32(b), ra, rb, sga, sgb)

        def step(i, carry):
            for b in (0, 1):
                ra, rb, et, sga, sgb, so = slots[b]
                kk = 2 * i + b

                @pl.when(kk < cnt)
                def _():
                    cb = (start + kk) * _CHUNK
                    off = kk * _CHUNK
                    pltpu.make_async_copy(
                        nh_hbm.at[idx_all.at[0, pl.ds(off, _CHUNK)]], ra, sga).wait()
                    pltpu.make_async_copy(
                        nh_hbm.at[idx_all.at[1, pl.ds(off, _CHUNK)]], rb, sgb).wait()

                    @pl.when(kk >= 2)
                    def _():
                        pltpu.make_async_copy(
                            et, out_hbm.at[:, pl.ds(cb, _CHUNK)], so).wait()

                    def compute(i16, c):
                        g16 = plsc.load_gather(
                            batch_v, [idx_all[0, pl.ds(off + i16 * 16, 16)]])
                        et[16, pl.ds(i16 * 16, 16)] = -(g16.astype(jnp.float32) + 0.25)
                        for j in range(16):
                            r = i16 * 16 + j
                            eh = jnp.maximum(ra[r, :], rb[r, :])
                            plsc.store_scatter(
                                et, [lanes, jnp.full((16,), r, jnp.int32)], eh)
                        return c
                    lax.fori_loop(0, _CHUNK // 16, compute, 0)

                    pltpu.async_copy(et, out_hbm.at[:, pl.ds(cb, _CHUNK)], so)

                    @pl.when(kk + 2 < cnt)
                    def _():
                        gath(kk + 2, ra, rb, sga, sgb)
            return carry
        lax.fori_loop(0, cpt // 2, step, 0)

        for b in (0, 1):
            _, _, et, _, _, so = slots[b]
            pltpu.make_async_copy(
                et, out_hbm.at[:, pl.ds(base, _CHUNK)], so).wait()

    return k(nh, ei, batch)


def _ecc_body(blk, y_ref, o_ref):
    yb = y_ref[0:16, :]
    gw = y_ref[16:17, :]
    yt = jnp.concatenate([yb] * _S, axis=0)
    srow = lax.broadcasted_iota(jnp.int32, (_ST, blk), 0) >> 4
    lin = srow.astype(jnp.float32) * jnp.float32(2.0 / 15.0) - 1.0
    sig = 0.5 + 0.5 * jnp.tanh(100.0 * (lin - yt))

    gbc = jnp.broadcast_to(gw, (_G, blk))
    iot = lax.broadcasted_iota(jnp.int32, (_G, blk), 0)
    oh = jnp.where(jnp.abs(gbc).astype(jnp.int32) == iot,
                   jnp.sign(gbc), 0.0)
    d = lax.dot_general(
        sig.astype(jnp.bfloat16), oh.astype(jnp.bfloat16),
        (((1,), (1,)), ((), ())), preferred_element_type=jnp.float32)

    @pl.when(pl.program_id(0) == 0)
    def _():
        o_ref[...] = d

    @pl.when(pl.program_id(0) != 0)
    def _():
        o_ref[...] += d


def _compute_ecc(y17, blk):
    n = y17.shape[1]
    return pl.pallas_call(
        functools.partial(_ecc_body, blk),
        grid=(n // blk,),
        in_specs=[pl.BlockSpec((_T + 1, blk), lambda i: (0, i))],
        out_specs=pl.BlockSpec((_ST, _G), lambda i: (0, 0)),
        out_shape=jax.ShapeDtypeStruct((_ST, _G), jnp.float32),
    )(y17)


def kernel(x, edge_index, batch, v):
    nh = _compute_nh(x, v)
    eh = _E // 2
    ea = _sc_edge_gather(nh, edge_index[:, :eh], batch, eh, 20, 19, 17)
    eb = _sc_edge_gather(nh, edge_index[:, eh:], batch, eh, 20, 19, 17)

    n17 = jnp.concatenate(
        [nh.T, (batch.astype(jnp.float32) + 0.25)[None, :]], axis=0)
    n17p = jnp.pad(n17, ((0, 0), (0, _NP - _N)))

    out2 = (_compute_ecc(n17p, _NODE_BLK)
            + _compute_ecc(ea, _EDGE_BLK)
            + _compute_ecc(eb, _EDGE_BLK))
    return out2.T.reshape(_G, _S, _T)
